# Initial kernel scaffold; baseline (speedup 1.0000x reference)
import jax
import jax.numpy as jnp
from jax.experimental import pallas as pl


def kernel(z, tgt, k):
    raise NotImplementedError("write your pallas kernel here")


# speedup vs baseline: 2.3479x; 2.3479x over previous
import functools

import jax
import jax.numpy as jnp
from jax import lax
from jax.experimental import pallas as pl
from jax.experimental.pallas import tpu as pltpu
from jax.experimental.pallas import tpu_sc as plsc

Q = 4096
KEYS = 65536
D = 768
TOPK = 4

QB = 512
KB = 2048

SC_CORES = 2
SC_SUBCORES = 16
SC_WORKERS = SC_CORES * SC_SUBCORES
GATHER_ROWS = Q * TOPK
ROWS_PER_WORKER = GATHER_ROWS // SC_WORKERS
CHUNK = 64
NCHUNKS = ROWS_PER_WORKER // CHUNK


def _topk_body(z_ref, tgt_ref, idx_ref, tn_s, rv_s, ri_s):
    kb = pl.program_id(0)
    qb = pl.program_id(1)
    nk = pl.num_programs(0)

    @pl.when(qb == 0)
    def _():
        t = tgt_ref[...]
        tn_s[...] = t * lax.rsqrt(jnp.sum(t * t, axis=-1, keepdims=True) + 1e-8)

    @pl.when(kb == 0)
    def _():
        rv_s[pl.ds(qb * QB, QB), :] = jnp.full((QB, TOPK), -jnp.inf, jnp.float32)
        ri_s[pl.ds(qb * QB, QB), :] = jnp.zeros((QB, TOPK), jnp.int32)

    z = z_ref[...]
    zn = z * lax.rsqrt(jnp.sum(z * z, axis=-1, keepdims=True) + 1e-8)
    sim = lax.dot_general(zn, tn_s[...], (((1,), (1,)), ((), ())),
                          preferred_element_type=jnp.float32)

    col = lax.broadcasted_iota(jnp.int32, (QB, KB), 1)
    base = kb * KB
    tvs, tis = [], []
    s = sim
    for _ in range(TOPK):
        m = jnp.max(s, axis=1, keepdims=True)
        pick = jnp.min(jnp.where(s == m, col, KEYS), axis=1, keepdims=True)
        tvs.append(m)
        tis.append(pick + base)
        s = jnp.where(col == pick, -jnp.inf, s)

    rv = rv_s[pl.ds(qb * QB, QB), :]
    ri = ri_s[pl.ds(qb * QB, QB), :]
    v8 = jnp.concatenate([rv] + tvs, axis=1)
    i8 = jnp.concatenate([ri] + tis, axis=1)
    col8 = lax.broadcasted_iota(jnp.int32, (QB, 2 * TOPK), 1)
    nvs, nis = [], []
    for _ in range(TOPK):
        m = jnp.max(v8, axis=1, keepdims=True)
        pos = jnp.min(jnp.where(v8 == m, col8, 2 * TOPK), axis=1, keepdims=True)
        sel = col8 == pos
        nvs.append(m)
        nis.append(jnp.sum(jnp.where(sel, i8, 0), axis=1, keepdims=True))
        v8 = jnp.where(sel, -jnp.inf, v8)
    rv_s[pl.ds(qb * QB, QB), :] = jnp.concatenate(nvs, axis=1)
    ri_new = jnp.concatenate(nis, axis=1)
    ri_s[pl.ds(qb * QB, QB), :] = ri_new

    @pl.when(kb == nk - 1)
    def _():
        idx_ref[...] = ri_new


def _topk_call(z, tgt):
    return pl.pallas_call(
        _topk_body,
        grid=(KEYS // KB, Q // QB),
        in_specs=[
            pl.BlockSpec((QB, D), lambda kb, qb: (qb, 0)),
            pl.BlockSpec((KB, D), lambda kb, qb: (kb, 0)),
        ],
        out_specs=pl.BlockSpec((QB, TOPK), lambda kb, qb: (qb, 0)),
        out_shape=jax.ShapeDtypeStruct((Q, TOPK), jnp.int32),
        scratch_shapes=[
            pltpu.VMEM((KB, D), jnp.float32),
            pltpu.VMEM((Q, TOPK), jnp.float32),
            pltpu.VMEM((Q, TOPK), jnp.int32),
        ],
    )(z, tgt)


def _gather_body(tgt_hbm, idx_hbm, out_hbm, idx_v, rows_v, sem):
    wid = lax.axis_index("s") * SC_CORES + lax.axis_index("c")
    base = wid * ROWS_PER_WORKER
    for c in range(NCHUNKS):
        off = base + c * CHUNK
        pltpu.sync_copy(idx_hbm.at[pl.ds(off, CHUNK)], idx_v)
        pltpu.async_copy(tgt_hbm.at[idx_v], rows_v, sem).wait()
        pltpu.sync_copy(rows_v, out_hbm.at[pl.ds(off, CHUNK)])


def _gather_call(tgt, idx_flat):
    fn = functools.partial(
        pl.kernel,
        mesh=plsc.VectorSubcoreMesh(core_axis_name="c", subcore_axis_name="s"),
        out_type=jax.ShapeDtypeStruct((GATHER_ROWS, D), jnp.float32),
        scratch_types=[
            pltpu.VMEM((CHUNK,), jnp.int32),
            pltpu.VMEM((CHUNK, D), jnp.float32),
            pltpu.SemaphoreType.DMA,
        ],
    )(_gather_body)
    return fn(tgt, idx_flat)


def _mean_body(g_ref, o_ref):
    g = g_ref[...]
    o_ref[...] = (g[:, :D] + g[:, D:2 * D] + g[:, 2 * D:3 * D] + g[:, 3 * D:]) * 0.25


def _mean_call(g2):
    return pl.pallas_call(
        _mean_body,
        grid=(Q // QB,),
        in_specs=[pl.BlockSpec((QB, TOPK * D), lambda i: (i, 0))],
        out_specs=pl.BlockSpec((QB, D), lambda i: (i, 0)),
        out_shape=jax.ShapeDtypeStruct((Q, D), jnp.float32),
    )(g2)


def kernel(z, tgt, k):
    del k
    idx = _topk_call(z, tgt)
    g = _gather_call(tgt, idx.reshape(GATHER_ROWS))
    return _mean_call(g.reshape(Q, TOPK * D))


# speedup vs baseline: 3.0530x; 1.3003x over previous
import functools

import jax
import jax.numpy as jnp
from jax import lax
from jax.experimental import pallas as pl
from jax.experimental.pallas import tpu as pltpu
from jax.experimental.pallas import tpu_sc as plsc

Q = 4096
KEYS = 65536
D = 768
TOPK = 4

QB = 512
KB = 2048

SC_CORES = 2
SC_SUBCORES = 16
SC_WORKERS = SC_CORES * SC_SUBCORES
GATHER_ROWS = Q * TOPK
ROWS_PER_WORKER = GATHER_ROWS // SC_WORKERS
CHUNK = 64
NCHUNKS = ROWS_PER_WORKER // CHUNK


def _topk_body(z_ref, tgt_ref, cv_ref, ci_ref, tn_s):
    qb = pl.program_id(1)

    @pl.when(qb == 0)
    def _():
        t = tgt_ref[...]
        tn_s[...] = t * lax.rsqrt(jnp.sum(t * t, axis=-1, keepdims=True) + 1e-8)

    z = z_ref[...]
    zn = z * lax.rsqrt(jnp.sum(z * z, axis=-1, keepdims=True) + 1e-8)
    sim = lax.dot_general(zn, tn_s[...], (((1,), (1,)), ((), ())),
                          preferred_element_type=jnp.float32)

    colf = lax.broadcasted_iota(jnp.int32, (QB, KB), 1).astype(jnp.float32)
    basef = (pl.program_id(0) * KB).astype(jnp.float32)
    tvs, tis = [], []
    s = sim
    for _ in range(TOPK):
        m = jnp.max(s, axis=1, keepdims=True)
        pick = jnp.min(jnp.where(s == m, colf, float(KEYS)), axis=1, keepdims=True)
        tvs.append(m)
        tis.append(pick + basef)
        s = jnp.where(colf == pick, -jnp.inf, s)

    cv_ref[0, :, :] = jnp.concatenate(tvs, axis=1)
    ci_ref[0, :, :] = jnp.concatenate(tis, axis=1)


def _topk_call(z, tgt):
    nkb = KEYS // KB
    return pl.pallas_call(
        _topk_body,
        grid=(nkb, Q // QB),
        in_specs=[
            pl.BlockSpec((QB, D), lambda kb, qb: (qb, 0)),
            pl.BlockSpec((KB, D), lambda kb, qb: (kb, 0)),
        ],
        out_specs=[
            pl.BlockSpec((1, QB, TOPK), lambda kb, qb: (kb, qb, 0)),
            pl.BlockSpec((1, QB, TOPK), lambda kb, qb: (kb, qb, 0)),
        ],
        out_shape=[
            jax.ShapeDtypeStruct((nkb, Q, TOPK), jnp.float32),
            jax.ShapeDtypeStruct((nkb, Q, TOPK), jnp.float32),
        ],
        scratch_shapes=[
            pltpu.VMEM((KB, D), jnp.float32),
        ],
    )(z, tgt)


def _merge_body(cv_ref, ci_ref, idx_ref):
    nc = (KEYS // KB) * TOPK
    v = cv_ref[...]
    gi = ci_ref[...]
    lanef = lax.broadcasted_iota(jnp.int32, (QB, nc), 1).astype(jnp.float32)
    outs = []
    for _ in range(TOPK):
        m = jnp.max(v, axis=1, keepdims=True)
        pos = jnp.min(jnp.where(v == m, lanef, float(nc)), axis=1, keepdims=True)
        sel = lanef == pos
        outs.append(jnp.sum(jnp.where(sel, gi, 0.0), axis=1, keepdims=True))
        v = jnp.where(sel, -jnp.inf, v)
    idx_ref[...] = jnp.concatenate(outs, axis=1).astype(jnp.int32)


def _merge_call(cv, ci):
    nc = (KEYS // KB) * TOPK
    cv2 = cv.transpose(1, 0, 2).reshape(Q, nc)
    ci2 = ci.transpose(1, 0, 2).reshape(Q, nc)
    return pl.pallas_call(
        _merge_body,
        grid=(Q // QB,),
        in_specs=[
            pl.BlockSpec((QB, nc), lambda q: (q, 0)),
            pl.BlockSpec((QB, nc), lambda q: (q, 0)),
        ],
        out_specs=pl.BlockSpec((QB, TOPK), lambda q: (q, 0)),
        out_shape=jax.ShapeDtypeStruct((Q, TOPK), jnp.int32),
    )(cv2, ci2)


def _gather_body(tgt_hbm, idx_hbm, out_hbm, idx_v, rows_v, sem):
    wid = lax.axis_index("s") * SC_CORES + lax.axis_index("c")
    base = wid * ROWS_PER_WORKER
    for c in range(NCHUNKS):
        off = base + c * CHUNK
        pltpu.sync_copy(idx_hbm.at[pl.ds(off, CHUNK)], idx_v)
        pltpu.async_copy(tgt_hbm.at[idx_v], rows_v, sem).wait()
        pltpu.sync_copy(rows_v, out_hbm.at[pl.ds(off, CHUNK)])


def _gather_call(tgt, idx_flat):
    fn = functools.partial(
        pl.kernel,
        mesh=plsc.VectorSubcoreMesh(core_axis_name="c", subcore_axis_name="s"),
        out_type=jax.ShapeDtypeStruct((GATHER_ROWS, D), jnp.float32),
        scratch_types=[
            pltpu.VMEM((CHUNK,), jnp.int32),
            pltpu.VMEM((CHUNK, D), jnp.float32),
            pltpu.SemaphoreType.DMA,
        ],
    )(_gather_body)
    return fn(tgt, idx_flat)


def _mean_body(g_ref, o_ref):
    g = g_ref[...]
    o_ref[...] = (g[:, :D] + g[:, D:2 * D] + g[:, 2 * D:3 * D] + g[:, 3 * D:]) * 0.25


def _mean_call(g2):
    return pl.pallas_call(
        _mean_body,
        grid=(Q // QB,),
        in_specs=[pl.BlockSpec((QB, TOPK * D), lambda i: (i, 0))],
        out_specs=pl.BlockSpec((QB, D), lambda i: (i, 0)),
        out_shape=jax.ShapeDtypeStruct((Q, D), jnp.float32),
    )(g2)


def kernel(z, tgt, k):
    del k
    cv, ci = _topk_call(z, tgt)
    idx = _merge_call(cv, ci)
    g = _gather_call(tgt, idx.reshape(GATHER_ROWS))
    return _mean_call(g.reshape(Q, TOPK * D))
